# baseline (speedup 1.0000x reference)
import jax
import jax.numpy as jnp
from jax.experimental import pallas as pl
from jax.experimental.pallas import tpu as pltpu

B, C, H, W = 64, 768, 24, 24
HID = 768


def _hat_cdf(t):
    tc = jnp.clip(t, -1.0, 1.0)
    return jnp.where(tc <= 0.0, 0.5 * (tc + 1.0) ** 2,
                     1.0 - 0.5 * (1.0 - tc) ** 2)


BB = 8


def _pool_kernel(box_ref, feat_ref, out_ref):
    b0 = pl.program_id(0) * BB
    w_f = jax.lax.broadcasted_iota(jnp.int32, (W, 128), 0).astype(jnp.float32)
    for j in range(BB):
        x0 = box_ref[b0 + j, 0] * W
        y0 = box_ref[b0 + j, 1] * W
        x1 = box_ref[b0 + j, 2] * W
        y1 = box_ref[b0 + j, 3] * W
        bw = (x1 - x0) * 0.25
        bh = (y1 - y0) * 0.25
        area = bw * bh
        scale = jnp.where(area > 0.0,
                          1.0 / (16.0 * jnp.maximum(area, 1e-12)), 0.0)
        vx = (_hat_cdf(x1 - w_f) - _hat_cdf(x0 - w_f)) * scale
        wx = pltpu.repeat(vx, C // 128, axis=1)
        x = feat_ref[j]
        acc = None
        for h in range(H):
            vy_h = _hat_cdf(y1 - float(h)) - _hat_cdf(y0 - float(h))
            term = x[h * W:(h + 1) * W, :] * vy_h
            acc = term if acc is None else acc + term
        acc = acc * wx
        s = acc[0:8, :] + acc[8:16, :] + acc[16:24, :]
        out_ref[j, 0, :] = jnp.sum(s, axis=0)


def _mlp_kernel(feat_ref, w1_ref, b1_ref, w2_ref, b2_ref, w3_ref, b3_ref,
                out_ref):
    dn = (((1,), (0,)), ((), ()))
    hi = jax.lax.Precision.HIGHEST
    x = feat_ref[...]
    h1 = jax.lax.dot_general(x, w1_ref[...], dn, precision=hi,
                             preferred_element_type=jnp.float32)
    h1 = jnp.maximum(h1 + b1_ref[...], 0.0)
    h2 = jax.lax.dot_general(h1, w2_ref[...], dn, precision=hi,
                             preferred_element_type=jnp.float32)
    h2 = jnp.maximum(h2 + b2_ref[...], 0.0)
    h3 = jax.lax.dot_general(h2, w3_ref[...], dn, precision=hi,
                             preferred_element_type=jnp.float32)
    out_ref[...] = h3 + b3_ref[...]


@jax.jit
def kernel(search_feat, search_box, w1, b1, w2, b2, w3, b3):
    pooled = pl.pallas_call(
        _pool_kernel,
        grid=(B // BB,),
        in_specs=[
            pl.BlockSpec(memory_space=pltpu.SMEM),
            pl.BlockSpec((BB, H * W, C), lambda b: (b, 0, 0)),
        ],
        out_specs=pl.BlockSpec((BB, 1, C), lambda b: (b, 0, 0)),
        out_shape=jax.ShapeDtypeStruct((B, 1, C), jnp.float32),
        compiler_params=pltpu.CompilerParams(
            dimension_semantics=("parallel",),
            vmem_limit_bytes=48 * 1024 * 1024,
        ),
    )(search_box, search_feat.transpose(0, 2, 3, 1).reshape(B, H * W, C))

    w3p = jnp.concatenate(
        [w3, jnp.zeros((HID, 127), jnp.float32)], axis=1)
    b3p = jnp.concatenate([b3, jnp.zeros((127,), jnp.float32)])

    out = pl.pallas_call(
        _mlp_kernel,
        out_shape=jax.ShapeDtypeStruct((B, 128), jnp.float32),
    )(pooled.reshape(B, C), w1, b1.reshape(1, HID), w2, b2.reshape(1, HID),
      w3p, b3p.reshape(1, 128))
    return out[:, :1].reshape(B, 1, 1)


# speedup vs baseline: 1.1174x; 1.1174x over previous
import jax
import jax.numpy as jnp
from jax.experimental import pallas as pl
from jax.experimental.pallas import tpu as pltpu

B, C, H, W = 64, 768, 24, 24
HID = 768


def _hat_cdf(t):
    tc = jnp.clip(t, -1.0, 1.0)
    return jnp.where(tc <= 0.0, 0.5 * (tc + 1.0) ** 2,
                     1.0 - 0.5 * (1.0 - tc) ** 2)


BB = 8


def _pool_kernel(box_ref, feat_ref, out_ref):
    b0 = pl.program_id(0) * BB
    w_f = jax.lax.broadcasted_iota(jnp.int32, (W, 128), 0).astype(jnp.float32)
    for j in range(BB):
        x0 = box_ref[0, b0 + j] * W
        y0 = box_ref[1, b0 + j] * W
        x1 = box_ref[2, b0 + j] * W
        y1 = box_ref[3, b0 + j] * W
        bw = (x1 - x0) * 0.25
        bh = (y1 - y0) * 0.25
        area = bw * bh
        scale = jnp.where(area > 0.0,
                          1.0 / (16.0 * jnp.maximum(area, 1e-12)), 0.0)
        vx = (_hat_cdf(x1 - w_f) - _hat_cdf(x0 - w_f)) * scale
        wx = pltpu.repeat(vx, C // 128, axis=1)
        x = feat_ref[j]
        acc = None
        for h in range(H):
            vy_h = _hat_cdf(y1 - float(h)) - _hat_cdf(y0 - float(h))
            term = x[h * W:(h + 1) * W, :] * vy_h
            acc = term if acc is None else acc + term
        acc = acc * wx
        s = acc[0:8, :] + acc[8:16, :] + acc[16:24, :]
        out_ref[j, 0, :] = jnp.sum(s, axis=0)


def _mlp_kernel(feat_ref, w1_ref, b1_ref, w2_ref, b2_ref, w3_ref, b3_ref,
                out_ref):
    dn = (((1,), (0,)), ((), ()))
    hi = jax.lax.Precision.HIGHEST
    x = feat_ref[...]
    h1 = jax.lax.dot_general(x, w1_ref[...], dn, precision=hi,
                             preferred_element_type=jnp.float32)
    h1 = jnp.maximum(h1 + b1_ref[...], 0.0)
    h2 = jax.lax.dot_general(h1, w2_ref[...], dn, precision=hi,
                             preferred_element_type=jnp.float32)
    h2 = jnp.maximum(h2 + b2_ref[...], 0.0)
    h3 = jax.lax.dot_general(w3_ref[...], h2, (((1,), (1,)), ((), ())),
                             precision=hi, preferred_element_type=jnp.float32)
    out_ref[...] = (h3 + b3_ref[0]).reshape(1, 1, B)


@jax.jit
def kernel(search_feat, search_box, w1, b1, w2, b2, w3, b3):
    pooled = pl.pallas_call(
        _pool_kernel,
        grid=(B // BB,),
        in_specs=[
            pl.BlockSpec(memory_space=pltpu.SMEM),
            pl.BlockSpec((BB, H * W, C), lambda b: (b, 0, 0)),
        ],
        out_specs=pl.BlockSpec((BB, 1, C), lambda b: (b, 0, 0)),
        out_shape=jax.ShapeDtypeStruct((B, 1, C), jnp.float32),
        compiler_params=pltpu.CompilerParams(
            dimension_semantics=("parallel",),
            vmem_limit_bytes=48 * 1024 * 1024,
        ),
    )(search_box.transpose(1, 0),
      search_feat.transpose(0, 2, 3, 1).reshape(B, H * W, C))

    n_in = 7
    out = pl.pallas_call(
        _mlp_kernel,
        in_specs=[pl.BlockSpec()] * (n_in - 1)
        + [pl.BlockSpec(memory_space=pltpu.SMEM)],
        out_shape=jax.ShapeDtypeStruct((1, 1, B), jnp.float32),
    )(pooled.reshape(B, C), w1, b1, w2, b2, w3.reshape(1, HID), b3)
    return out.reshape(B, 1, 1)
